# baseline (speedup 1.0000x reference)
import functools

import jax
import jax.numpy as jnp
from jax import lax
from jax.experimental import pallas as pl
from jax.experimental.pallas import tpu as pltpu
from jax.experimental.pallas import tpu_sc as plsc

D_MODEL_K = 1024
VOCAB_K = 100000
SCALE = float(D_MODEL_K) ** 0.5

NW = 32
B_TOTAL = 4 * 4096
B_PER_W = B_TOTAL // NW
CHUNK = 32
NCHUNK = B_PER_W // CHUNK
LANES = 16


@functools.partial(
    pl.kernel,
    out_type=jax.ShapeDtypeStruct((B_TOTAL, D_MODEL_K), jnp.float32),
    mesh=plsc.VectorSubcoreMesh(core_axis_name="c", subcore_axis_name="s"),
    scratch_types=[
        pltpu.VMEM((NCHUNK, CHUNK), jnp.int32),
        pltpu.VMEM((CHUNK, D_MODEL_K), jnp.float32),
        pltpu.SemaphoreType.DMA,
    ],
)
def _emb_lookup(x_hbm, table_hbm, out_hbm, idx_v, rows_v, sem):
    cid = lax.axis_index("c")
    sid = lax.axis_index("s")
    wid = sid * 2 + cid
    base = wid * B_PER_W
    pltpu.sync_copy(x_hbm.at[wid], idx_v)
    for g in range(NCHUNK):
        pltpu.async_copy(table_hbm.at[idx_v.at[g]], rows_v, sem).wait()

        def scale_row(r, _):
            def scale_vec(j, _):
                sl = pl.ds(j * LANES, LANES)
                rows_v[r, sl] = rows_v[r, sl] * SCALE
                return 0

            return lax.fori_loop(0, D_MODEL_K // LANES, scale_vec, 0)

        lax.fori_loop(0, CHUNK, scale_row, 0)
        pltpu.sync_copy(rows_v, out_hbm.at[pl.ds(base + g * CHUNK, CHUNK)])


def kernel(x, table):
    xr = x.reshape(NW, NCHUNK, CHUNK)
    out = _emb_lookup(xr, table)
    return out.reshape(4, 4096, D_MODEL_K)


# speedup vs baseline: 3.3127x; 3.3127x over previous
import functools

import jax
import jax.numpy as jnp
from jax import lax
from jax.experimental import pallas as pl
from jax.experimental.pallas import tpu as pltpu
from jax.experimental.pallas import tpu_sc as plsc

D_MODEL_K = 1024
SCALE = float(D_MODEL_K) ** 0.5

NW = 32
B_TOTAL = 4 * 4096
B_PER_W = B_TOTAL // NW
CHUNK = 32
NCHUNK = B_PER_W // CHUNK
NBUF = 3
LANES = 16
VPR = D_MODEL_K // LANES


@functools.partial(
    pl.kernel,
    out_type=jax.ShapeDtypeStruct((B_TOTAL, D_MODEL_K), jnp.float32),
    mesh=plsc.VectorSubcoreMesh(core_axis_name="c", subcore_axis_name="s"),
    scratch_types=(
        [pltpu.VMEM((NCHUNK, CHUNK), jnp.int32)]
        + [pltpu.VMEM((CHUNK, D_MODEL_K), jnp.float32) for _ in range(NBUF)]
        + [pltpu.SemaphoreType.DMA for _ in range(2 * NBUF)]
    ),
)
def _emb_lookup(x_hbm, table_hbm, out_hbm, idx_v, b0, b1, b2,
                g0, g1, g2, s0, s1, s2):
    bufs = (b0, b1, b2)
    gsem = (g0, g1, g2)
    ssem = (s0, s1, s2)
    cid = lax.axis_index("c")
    sid = lax.axis_index("s")
    wid = sid * 2 + cid
    base = wid * B_PER_W
    pltpu.sync_copy(x_hbm.at[wid], idx_v)

    def start_gather(g):
        b = g % NBUF
        return pltpu.async_copy(table_hbm.at[idx_v.at[g]], bufs[b], gsem[b])

    def scale_buf(b):
        rows = bufs[b]

        def scale_row(r, _):
            for j in range(VPR):
                sl = pl.ds(j * LANES, LANES)
                rows[r, sl] = rows[r, sl] * SCALE
            return 0

        lax.fori_loop(0, CHUNK, scale_row, 0)

    gh = {0: start_gather(0)}
    sh = {}
    for g in range(NCHUNK):
        b = g % NBUF
        if g + 1 < NCHUNK:
            if g - 2 >= 0:
                sh.pop(g - 2).wait()
            gh[g + 1] = start_gather(g + 1)
        gh.pop(g).wait()
        scale_buf(b)
        sh[g] = pltpu.async_copy(
            bufs[b], out_hbm.at[pl.ds(base + g * CHUNK, CHUNK)], ssem[b])
    for g in sorted(sh):
        sh.pop(g).wait()


def kernel(x, table):
    xr = x.reshape(NW, NCHUNK, CHUNK)
    out = _emb_lookup(xr, table)
    return out.reshape(4, 4096, D_MODEL_K)
